# baseline (speedup 1.0000x reference)
import math

import jax
import jax.numpy as jnp
from jax.experimental import pallas as pl

N = 4096
BLK = 256
NBLK = N // BLK
KDIM = 256
KNN = 32
PIX = 256
NIMG = 16

_INTERPRET = False


def _affinity_body(a_ref, b_ref, res_ref):
    i = pl.program_id(0)
    s = jax.lax.dot_general(a_ref[...], b_ref[...], (((1,), (1,)), ((), ())),
                            preferred_element_type=jnp.float32)
    s = jnp.maximum(s, 0.0)
    rows_g = jax.lax.broadcasted_iota(jnp.int32, (BLK, N), 0) + i * BLK
    cols = jax.lax.broadcasted_iota(jnp.int32, (BLK, N), 1)
    s = jnp.where(rows_g == cols, 0.0, s)

    ones = jnp.ones((N, 128), jnp.bfloat16)

    def bisect(_, carry):
        lo, hi = carry
        mid = (lo + hi) * 0.5
        ind = jnp.where(s >= mid, 1.0, 0.0).astype(jnp.bfloat16)
        cnt = jax.lax.dot_general(ind, ones, (((1,), (0,)), ((), ())),
                                  preferred_element_type=jnp.float32)[:, 0:1]
        ge = cnt >= float(KNN)
        return jnp.where(ge, mid, lo), jnp.where(ge, hi, mid)

    lo = jnp.zeros((BLK, 1), jnp.float32)
    hi = jnp.max(s, axis=1, keepdims=True) * 1.0000002 + 1e-30
    lo, hi = jax.lax.fori_loop(0, 34, bisect, (lo, hi))
    res_ref[...] = jnp.where(s >= lo, s, 0.0)


def _symmetrize_body(res_rb_ref, res_cb_ref, asym_ref, s_ref):
    asym = (res_rb_ref[...] + jnp.transpose(res_cb_ref[...])) / 2.0
    asym_ref[...] = asym
    s_ref[...] = jnp.sum(asym, axis=1, keepdims=True)


def _pixel_body(frow_ref, fcol_ref, g2_ref):
    frow = frow_ref[0]
    fcol = fcol_ref[0]
    ri = jax.lax.broadcasted_iota(jnp.int32, (PIX, 1), 0)
    ci_full = jax.lax.broadcasted_iota(jnp.int32, (PIX, PIX), 1)
    sq_rgb = None
    for c in range(3):
        diff = fcol[:, c:c + 1] - frow[c:c + 1, :]
        t = diff * diff
        sq_rgb = t if sq_rgb is None else sq_rgb + t

    adj = jnp.zeros((PIX, PIX), jnp.float32)
    for k, dw in ((10, 2.0), (5, 0.1)):
        dx = fcol[:, 3:4] * dw - frow[3:4, :] * dw
        dy = fcol[:, 4:5] * dw - frow[4:5, :] * dw
        sq = (sq_rgb + dx * dx) + dy * dy
        euc = -jnp.sqrt(jnp.maximum(sq, 0.0))
        euc = jnp.where(ri == ci_full, -jnp.inf, euc)
        work = euc
        for _ in range(k):
            m = jnp.max(work, axis=1, keepdims=True)
            cand = jnp.where(work == m, ci_full, N)
            j = jnp.min(cand, axis=1, keepdims=True)
            hit = ci_full == j
            adj = jnp.maximum(adj, hit.astype(jnp.float32))
            work = jnp.where(hit, -jnp.inf, work)
    eye = (ri == ci_full).astype(jnp.float32)
    adj_t = jax.lax.dot_general(adj, eye, (((0,), (0,)), ((), ())),
                                preferred_element_type=jnp.float32)
    adj = jnp.maximum(adj, adj_t)
    deg_col = jnp.sum(adj, axis=1, keepdims=True)
    deg_row = jnp.sum(adj, axis=0, keepdims=True)
    d2c = 1.0 / jnp.sqrt(deg_col)
    d2r = 1.0 / jnp.sqrt(deg_row)
    g2_ref[0] = (adj * d2r) * d2c


def _t1_body(asym_ref, sj_ref, srow_ref, g2_ref, psi_ref, t1t_ref):
    j = pl.program_id(0)
    ci = pl.program_id(1)
    dcol = 1.0 / jnp.sqrt(sj_ref[...])
    drow = 1.0 / jnp.sqrt(srow_ref[0:1, :])
    g = (asym_ref[...] * dcol) * drow
    ondiag = jnp.where(ci == j, 1.0, 0.0)
    g = g + (g2_ref[0] * 0.3) * ondiag

    @pl.when(ci == 0)
    def _():
        t1t_ref[...] = jnp.zeros_like(t1t_ref)

    t1t_ref[...] += jax.lax.dot_general(g, psi_ref[...],
                                        (((1,), (0,)), ((), ())),
                                        preferred_element_type=jnp.float32)


def _final_body(t1t_ref, psi_ref, loss_ref, reg_ref):
    r = jax.lax.dot_general(t1t_ref[...], psi_ref[...],
                            (((0,), (0,)), ((), ())),
                            preferred_element_type=jnp.float32)
    ri = jax.lax.broadcasted_iota(jnp.int32, (KDIM, KDIM), 0)
    ci = jax.lax.broadcasted_iota(jnp.int32, (KDIM, KDIM), 1)
    eye = (ri == ci).astype(jnp.float32)
    diag_part = jnp.sum(r * eye, axis=1, keepdims=True)
    loss_ref[...] = -jnp.sum(diag_part, axis=0, keepdims=True) / float(KDIM)
    upper = jnp.where(ci > ri, r * r, 0.0)
    reg_ref[...] = jnp.sum(jnp.sum(upper, axis=1, keepdims=True), axis=0,
                           keepdims=True) * (0.05 / float(KDIM))


def kernel(highlevel_feature, Psi, im):
    hf = highlevel_feature.reshape(-1, highlevel_feature.shape[-1])
    fdim = hf.shape[1]
    hfn = hf / jnp.maximum(jnp.linalg.norm(hf, axis=-1, keepdims=True), 1e-12)
    psi_s = Psi.reshape(-1, Psi.shape[-1]).astype(jnp.float32) * math.sqrt(10.0)

    res = pl.pallas_call(
        _affinity_body,
        grid=(NBLK,),
        in_specs=[
            pl.BlockSpec((BLK, fdim), lambda i: (i, 0)),
            pl.BlockSpec((N, fdim), lambda i: (0, 0)),
        ],
        out_specs=pl.BlockSpec((BLK, N), lambda i: (i, 0)),
        out_shape=jax.ShapeDtypeStruct((N, N), jnp.float32),
        interpret=_INTERPRET,
    )(hfn, hfn)

    asym, s = pl.pallas_call(
        _symmetrize_body,
        grid=(NBLK,),
        in_specs=[
            pl.BlockSpec((BLK, N), lambda i: (i, 0)),
            pl.BlockSpec((N, BLK), lambda i: (0, i)),
        ],
        out_specs=[
            pl.BlockSpec((BLK, N), lambda i: (i, 0)),
            pl.BlockSpec((BLK, 1), lambda i: (i, 0)),
        ],
        out_shape=[
            jax.ShapeDtypeStruct((N, N), jnp.float32),
            jax.ShapeDtypeStruct((N, 1), jnp.float32),
        ],
        interpret=_INTERPRET,
    )(res, res)

    rgb = ((im + 1.0) / 2.0).reshape(NIMG, 3, PIX)
    x_ = jnp.tile(jnp.linspace(0.0, 1.0, 16), (16,)).astype(jnp.float32)
    y_ = jnp.repeat(jnp.linspace(0.0, 1.0, 16), 16).astype(jnp.float32)
    frow = jnp.concatenate(
        [rgb, jnp.broadcast_to(x_.reshape(1, 1, PIX), (NIMG, 1, PIX)),
         jnp.broadcast_to(y_.reshape(1, 1, PIX), (NIMG, 1, PIX))], axis=1)
    fcol = jnp.zeros((NIMG, PIX, 8), jnp.float32)
    fcol = fcol.at[:, :, 0:3].set(rgb.transpose(0, 2, 1))
    fcol = fcol.at[:, :, 3].set(x_[None, :])
    fcol = fcol.at[:, :, 4].set(y_[None, :])

    g2 = pl.pallas_call(
        _pixel_body,
        grid=(NIMG,),
        in_specs=[
            pl.BlockSpec((1, 5, PIX), lambda b: (b, 0, 0)),
            pl.BlockSpec((1, PIX, 8), lambda b: (b, 0, 0)),
        ],
        out_specs=pl.BlockSpec((1, PIX, PIX), lambda b: (b, 0, 0)),
        out_shape=jax.ShapeDtypeStruct((NIMG, PIX, PIX), jnp.float32),
        interpret=_INTERPRET,
    )(frow, fcol)

    s_row8 = jnp.broadcast_to(s.reshape(1, N), (8, N))

    t1t = pl.pallas_call(
        _t1_body,
        grid=(NBLK, NBLK),
        in_specs=[
            pl.BlockSpec((BLK, BLK), lambda j, ci: (j, ci)),
            pl.BlockSpec((BLK, 1), lambda j, ci: (j, 0)),
            pl.BlockSpec((8, BLK), lambda j, ci: (0, ci)),
            pl.BlockSpec((1, PIX, PIX), lambda j, ci: (j, 0, 0)),
            pl.BlockSpec((BLK, KDIM), lambda j, ci: (ci, 0)),
        ],
        out_specs=pl.BlockSpec((BLK, KDIM), lambda j, ci: (j, 0)),
        out_shape=jax.ShapeDtypeStruct((N, KDIM), jnp.float32),
        interpret=_INTERPRET,
    )(asym, s, s_row8, g2, psi_s)

    loss, reg = pl.pallas_call(
        _final_body,
        grid=(1,),
        in_specs=[
            pl.BlockSpec((N, KDIM), lambda i: (0, 0)),
            pl.BlockSpec((N, KDIM), lambda i: (0, 0)),
        ],
        out_specs=[
            pl.BlockSpec((1, 1), lambda i: (0, 0)),
            pl.BlockSpec((1, 1), lambda i: (0, 0)),
        ],
        out_shape=[
            jax.ShapeDtypeStruct((1, 1), jnp.float32),
            jax.ShapeDtypeStruct((1, 1), jnp.float32),
        ],
        interpret=_INTERPRET,
    )(t1t, psi_s)

    return (loss.reshape(()), reg.reshape(()))


# speedup vs baseline: 1.1949x; 1.1949x over previous
import math

import jax
import jax.numpy as jnp
from jax.experimental import pallas as pl

N = 4096
BLK = 256
NBLK = N // BLK
KDIM = 256
KNN = 32
PIX = 256
NIMG = 16

_INTERPRET = False


def _affinity_body(a_ref, b_ref, res_ref):
    i = pl.program_id(0)
    s = jax.lax.dot_general(a_ref[...], b_ref[...], (((1,), (1,)), ((), ())),
                            preferred_element_type=jnp.float32)
    s = jnp.maximum(s, 0.0)
    rows_g = jax.lax.broadcasted_iota(jnp.int32, (BLK, N), 0) + i * BLK
    cols = jax.lax.broadcasted_iota(jnp.int32, (BLK, N), 1)
    s = jnp.where(rows_g == cols, 0.0, s)

    def bisect(_, carry):
        lo, hi = carry
        mid = (lo + hi) * 0.5
        cnt = jnp.sum((s >= mid).astype(jnp.float32), axis=1, keepdims=True)
        ge = cnt >= float(KNN)
        return jnp.where(ge, mid, lo), jnp.where(ge, hi, mid)

    lo = jnp.zeros((BLK, 1), jnp.float32)
    hi = jnp.max(s, axis=1, keepdims=True) * 1.0000002 + 1e-30
    lo, hi = jax.lax.fori_loop(0, 34, bisect, (lo, hi))
    res_ref[...] = jnp.where(s >= lo, s, 0.0)


def _symmetrize_body(res_rb_ref, res_cb_ref, asym_ref, s_ref):
    asym = (res_rb_ref[...] + jnp.transpose(res_cb_ref[...])) / 2.0
    asym_ref[...] = asym
    s_ref[...] = jnp.sum(asym, axis=1, keepdims=True)


def _pixel_body(frow_ref, fcol_ref, g2_ref):
    frow = frow_ref[0]
    fcol = fcol_ref[0]
    ri = jax.lax.broadcasted_iota(jnp.int32, (PIX, 1), 0)
    ci_full = jax.lax.broadcasted_iota(jnp.int32, (PIX, PIX), 1)
    sq_rgb = None
    for c in range(3):
        diff = fcol[:, c:c + 1] - frow[c:c + 1, :]
        t = diff * diff
        sq_rgb = t if sq_rgb is None else sq_rgb + t

    adj = jnp.zeros((PIX, PIX), jnp.float32)
    for k, dw in ((10, 2.0), (5, 0.1)):
        dx = fcol[:, 3:4] * dw - frow[3:4, :] * dw
        dy = fcol[:, 4:5] * dw - frow[4:5, :] * dw
        sq = (sq_rgb + dx * dx) + dy * dy
        euc = -jnp.sqrt(jnp.maximum(sq, 0.0))
        euc = jnp.where(ri == ci_full, -jnp.inf, euc)
        work = euc
        for _ in range(k):
            m = jnp.max(work, axis=1, keepdims=True)
            cand = jnp.where(work == m, ci_full, N)
            j = jnp.min(cand, axis=1, keepdims=True)
            hit = ci_full == j
            adj = jnp.maximum(adj, hit.astype(jnp.float32))
            work = jnp.where(hit, -jnp.inf, work)
    eye = (ri == ci_full).astype(jnp.float32)
    adj_t = jax.lax.dot_general(adj, eye, (((0,), (0,)), ((), ())),
                                preferred_element_type=jnp.float32)
    adj = jnp.maximum(adj, adj_t)
    deg_col = jnp.sum(adj, axis=1, keepdims=True)
    deg_row = jnp.sum(adj, axis=0, keepdims=True)
    d2c = 1.0 / jnp.sqrt(deg_col)
    d2r = 1.0 / jnp.sqrt(deg_row)
    g2_ref[0] = (adj * d2r) * d2c


def _t1_body(asym_ref, sj_ref, srow_ref, g2_ref, psi_ref, t1t_ref):
    j = pl.program_id(0)
    ci = pl.program_id(1)
    dcol = 1.0 / jnp.sqrt(sj_ref[...])
    drow = 1.0 / jnp.sqrt(srow_ref[0:1, :])
    g = (asym_ref[...] * dcol) * drow
    ondiag = jnp.where(ci == j, 1.0, 0.0)
    g = g + (g2_ref[0] * 0.3) * ondiag

    @pl.when(ci == 0)
    def _():
        t1t_ref[...] = jnp.zeros_like(t1t_ref)

    t1t_ref[...] += jax.lax.dot_general(g, psi_ref[...],
                                        (((1,), (0,)), ((), ())),
                                        preferred_element_type=jnp.float32)


def _final_body(t1t_ref, psi_ref, loss_ref, reg_ref):
    r = jax.lax.dot_general(t1t_ref[...], psi_ref[...],
                            (((0,), (0,)), ((), ())),
                            preferred_element_type=jnp.float32)
    ri = jax.lax.broadcasted_iota(jnp.int32, (KDIM, KDIM), 0)
    ci = jax.lax.broadcasted_iota(jnp.int32, (KDIM, KDIM), 1)
    eye = (ri == ci).astype(jnp.float32)
    diag_part = jnp.sum(r * eye, axis=1, keepdims=True)
    loss_ref[...] = -jnp.sum(diag_part, axis=0, keepdims=True) / float(KDIM)
    upper = jnp.where(ci > ri, r * r, 0.0)
    reg_ref[...] = jnp.sum(jnp.sum(upper, axis=1, keepdims=True), axis=0,
                           keepdims=True) * (0.05 / float(KDIM))


def kernel(highlevel_feature, Psi, im):
    hf = highlevel_feature.reshape(-1, highlevel_feature.shape[-1])
    fdim = hf.shape[1]
    hfn = hf / jnp.maximum(jnp.linalg.norm(hf, axis=-1, keepdims=True), 1e-12)
    psi_s = Psi.reshape(-1, Psi.shape[-1]).astype(jnp.float32) * math.sqrt(10.0)

    res = pl.pallas_call(
        _affinity_body,
        grid=(NBLK,),
        in_specs=[
            pl.BlockSpec((BLK, fdim), lambda i: (i, 0)),
            pl.BlockSpec((N, fdim), lambda i: (0, 0)),
        ],
        out_specs=pl.BlockSpec((BLK, N), lambda i: (i, 0)),
        out_shape=jax.ShapeDtypeStruct((N, N), jnp.float32),
        interpret=_INTERPRET,
    )(hfn, hfn)

    asym, s = pl.pallas_call(
        _symmetrize_body,
        grid=(NBLK,),
        in_specs=[
            pl.BlockSpec((BLK, N), lambda i: (i, 0)),
            pl.BlockSpec((N, BLK), lambda i: (0, i)),
        ],
        out_specs=[
            pl.BlockSpec((BLK, N), lambda i: (i, 0)),
            pl.BlockSpec((BLK, 1), lambda i: (i, 0)),
        ],
        out_shape=[
            jax.ShapeDtypeStruct((N, N), jnp.float32),
            jax.ShapeDtypeStruct((N, 1), jnp.float32),
        ],
        interpret=_INTERPRET,
    )(res, res)

    rgb = ((im + 1.0) / 2.0).reshape(NIMG, 3, PIX)
    x_ = jnp.tile(jnp.linspace(0.0, 1.0, 16), (16,)).astype(jnp.float32)
    y_ = jnp.repeat(jnp.linspace(0.0, 1.0, 16), 16).astype(jnp.float32)
    frow = jnp.concatenate(
        [rgb, jnp.broadcast_to(x_.reshape(1, 1, PIX), (NIMG, 1, PIX)),
         jnp.broadcast_to(y_.reshape(1, 1, PIX), (NIMG, 1, PIX))], axis=1)
    fcol = jnp.zeros((NIMG, PIX, 8), jnp.float32)
    fcol = fcol.at[:, :, 0:3].set(rgb.transpose(0, 2, 1))
    fcol = fcol.at[:, :, 3].set(x_[None, :])
    fcol = fcol.at[:, :, 4].set(y_[None, :])

    g2 = pl.pallas_call(
        _pixel_body,
        grid=(NIMG,),
        in_specs=[
            pl.BlockSpec((1, 5, PIX), lambda b: (b, 0, 0)),
            pl.BlockSpec((1, PIX, 8), lambda b: (b, 0, 0)),
        ],
        out_specs=pl.BlockSpec((1, PIX, PIX), lambda b: (b, 0, 0)),
        out_shape=jax.ShapeDtypeStruct((NIMG, PIX, PIX), jnp.float32),
        interpret=_INTERPRET,
    )(frow, fcol)

    s_row8 = jnp.broadcast_to(s.reshape(1, N), (8, N))

    t1t = pl.pallas_call(
        _t1_body,
        grid=(NBLK, NBLK),
        in_specs=[
            pl.BlockSpec((BLK, BLK), lambda j, ci: (j, ci)),
            pl.BlockSpec((BLK, 1), lambda j, ci: (j, 0)),
            pl.BlockSpec((8, BLK), lambda j, ci: (0, ci)),
            pl.BlockSpec((1, PIX, PIX), lambda j, ci: (j, 0, 0)),
            pl.BlockSpec((BLK, KDIM), lambda j, ci: (ci, 0)),
        ],
        out_specs=pl.BlockSpec((BLK, KDIM), lambda j, ci: (j, 0)),
        out_shape=jax.ShapeDtypeStruct((N, KDIM), jnp.float32),
        interpret=_INTERPRET,
    )(asym, s, s_row8, g2, psi_s)

    loss, reg = pl.pallas_call(
        _final_body,
        grid=(1,),
        in_specs=[
            pl.BlockSpec((N, KDIM), lambda i: (0, 0)),
            pl.BlockSpec((N, KDIM), lambda i: (0, 0)),
        ],
        out_specs=[
            pl.BlockSpec((1, 1), lambda i: (0, 0)),
            pl.BlockSpec((1, 1), lambda i: (0, 0)),
        ],
        out_shape=[
            jax.ShapeDtypeStruct((1, 1), jnp.float32),
            jax.ShapeDtypeStruct((1, 1), jnp.float32),
        ],
        interpret=_INTERPRET,
    )(t1t, psi_s)

    return (loss.reshape(()), reg.reshape(()))


# speedup vs baseline: 1.2593x; 1.0539x over previous
import math

import jax
import jax.numpy as jnp
from jax.experimental import pallas as pl

N = 4096
BLK = 256
NBLK = N // BLK
KDIM = 256
KNN = 32
PIX = 256
NIMG = 16

_INTERPRET = False


def _affinity_body(a_ref, b_ref, res_ref, rowsum_ref, colsum_ref):
    i = pl.program_id(0)
    s = jax.lax.dot_general(a_ref[...], b_ref[...], (((1,), (1,)), ((), ())),
                            preferred_element_type=jnp.float32)
    s = jnp.maximum(s, 0.0)
    rows_g = jax.lax.broadcasted_iota(jnp.int32, (BLK, N), 0) + i * BLK
    cols = jax.lax.broadcasted_iota(jnp.int32, (BLK, N), 1)
    s = jnp.where(rows_g == cols, 0.0, s)

    def bisect(_, carry):
        lo, hi = carry
        mid = (lo + hi) * 0.5
        cnt = jnp.sum((s >= mid).astype(jnp.float32), axis=1, keepdims=True)
        ge = cnt >= float(KNN)
        return jnp.where(ge, mid, lo), jnp.where(ge, hi, mid)

    lo = jnp.zeros((BLK, 1), jnp.float32)
    hi = jnp.max(s, axis=1, keepdims=True) * 1.0000002 + 1e-30
    lo, hi = jax.lax.fori_loop(0, 34, bisect, (lo, hi))
    res = jnp.where(s >= lo, s, 0.0)
    res_ref[...] = res
    rowsum_ref[...] = jnp.sum(res, axis=1, keepdims=True)

    @pl.when(i == 0)
    def _():
        colsum_ref[...] = jnp.zeros_like(colsum_ref)

    colsum_ref[0:1, :] += jnp.sum(res, axis=0, keepdims=True)


def _pixel_body(frow_ref, fcol_ref, g2_ref):
    frow = frow_ref[0]
    fcol = fcol_ref[0]
    ri = jax.lax.broadcasted_iota(jnp.int32, (PIX, 1), 0)
    ci_full = jax.lax.broadcasted_iota(jnp.int32, (PIX, PIX), 1)
    sq_rgb = None
    for c in range(3):
        diff = fcol[:, c:c + 1] - frow[c:c + 1, :]
        t = diff * diff
        sq_rgb = t if sq_rgb is None else sq_rgb + t

    adj = jnp.zeros((PIX, PIX), jnp.float32)
    for k, dw in ((10, 2.0), (5, 0.1)):
        dx = fcol[:, 3:4] * dw - frow[3:4, :] * dw
        dy = fcol[:, 4:5] * dw - frow[4:5, :] * dw
        sq = (sq_rgb + dx * dx) + dy * dy
        euc = -jnp.sqrt(jnp.maximum(sq, 0.0))
        euc = jnp.where(ri == ci_full, -jnp.inf, euc)
        work = euc
        for _ in range(k):
            m = jnp.max(work, axis=1, keepdims=True)
            cand = jnp.where(work == m, ci_full, N)
            j = jnp.min(cand, axis=1, keepdims=True)
            hit = ci_full == j
            adj = jnp.maximum(adj, hit.astype(jnp.float32))
            work = jnp.where(hit, -jnp.inf, work)
    eye = (ri == ci_full).astype(jnp.float32)
    adj_t = jax.lax.dot_general(adj, eye, (((0,), (0,)), ((), ())),
                                preferred_element_type=jnp.float32)
    adj = jnp.maximum(adj, adj_t)
    deg_col = jnp.sum(adj, axis=1, keepdims=True)
    deg_row = jnp.sum(adj, axis=0, keepdims=True)
    d2c = 1.0 / jnp.sqrt(deg_col)
    d2r = 1.0 / jnp.sqrt(deg_row)
    g2_ref[0] = (adj * d2r) * d2c


def _t1_body(res_jc_ref, res_cj_ref, sj_ref, srow_ref, g2_ref, psi_ref, t1t_ref):
    j = pl.program_id(0)
    ci = pl.program_id(1)
    asym = (res_jc_ref[...] + jnp.transpose(res_cj_ref[...])) / 2.0
    dcol = 1.0 / jnp.sqrt(sj_ref[...])
    drow = 1.0 / jnp.sqrt(srow_ref[0:1, :])
    g = (asym * dcol) * drow
    ondiag = jnp.where(ci == j, 1.0, 0.0)
    g = g + (g2_ref[0] * 0.3) * ondiag

    @pl.when(ci == 0)
    def _():
        t1t_ref[...] = jnp.zeros_like(t1t_ref)

    t1t_ref[...] += jax.lax.dot_general(g, psi_ref[...],
                                        (((1,), (0,)), ((), ())),
                                        preferred_element_type=jnp.float32)


def _final_body(t1t_ref, psi_ref, loss_ref, reg_ref):
    r = jax.lax.dot_general(t1t_ref[...], psi_ref[...],
                            (((0,), (0,)), ((), ())),
                            preferred_element_type=jnp.float32)
    ri = jax.lax.broadcasted_iota(jnp.int32, (KDIM, KDIM), 0)
    ci = jax.lax.broadcasted_iota(jnp.int32, (KDIM, KDIM), 1)
    eye = (ri == ci).astype(jnp.float32)
    diag_part = jnp.sum(r * eye, axis=1, keepdims=True)
    loss_ref[...] = -jnp.sum(diag_part, axis=0, keepdims=True) / float(KDIM)
    upper = jnp.where(ci > ri, r * r, 0.0)
    reg_ref[...] = jnp.sum(jnp.sum(upper, axis=1, keepdims=True), axis=0,
                           keepdims=True) * (0.05 / float(KDIM))


def kernel(highlevel_feature, Psi, im):
    hf = highlevel_feature.reshape(-1, highlevel_feature.shape[-1])
    fdim = hf.shape[1]
    hfn = hf / jnp.maximum(jnp.linalg.norm(hf, axis=-1, keepdims=True), 1e-12)
    psi_s = Psi.reshape(-1, Psi.shape[-1]).astype(jnp.float32) * math.sqrt(10.0)

    res, rowsum, colsum = pl.pallas_call(
        _affinity_body,
        grid=(NBLK,),
        in_specs=[
            pl.BlockSpec((BLK, fdim), lambda i: (i, 0)),
            pl.BlockSpec((N, fdim), lambda i: (0, 0)),
        ],
        out_specs=[
            pl.BlockSpec((BLK, N), lambda i: (i, 0)),
            pl.BlockSpec((BLK, 1), lambda i: (i, 0)),
            pl.BlockSpec((8, N), lambda i: (0, 0)),
        ],
        out_shape=[
            jax.ShapeDtypeStruct((N, N), jnp.float32),
            jax.ShapeDtypeStruct((N, 1), jnp.float32),
            jax.ShapeDtypeStruct((8, N), jnp.float32),
        ],
        interpret=_INTERPRET,
    )(hfn, hfn)

    s = (rowsum + colsum[0:1, :].reshape(N, 1)) / 2.0

    rgb = ((im + 1.0) / 2.0).reshape(NIMG, 3, PIX)
    x_ = jnp.tile(jnp.linspace(0.0, 1.0, 16), (16,)).astype(jnp.float32)
    y_ = jnp.repeat(jnp.linspace(0.0, 1.0, 16), 16).astype(jnp.float32)
    frow = jnp.concatenate(
        [rgb, jnp.broadcast_to(x_.reshape(1, 1, PIX), (NIMG, 1, PIX)),
         jnp.broadcast_to(y_.reshape(1, 1, PIX), (NIMG, 1, PIX))], axis=1)
    fcol = jnp.zeros((NIMG, PIX, 8), jnp.float32)
    fcol = fcol.at[:, :, 0:3].set(rgb.transpose(0, 2, 1))
    fcol = fcol.at[:, :, 3].set(x_[None, :])
    fcol = fcol.at[:, :, 4].set(y_[None, :])

    g2 = pl.pallas_call(
        _pixel_body,
        grid=(NIMG,),
        in_specs=[
            pl.BlockSpec((1, 5, PIX), lambda b: (b, 0, 0)),
            pl.BlockSpec((1, PIX, 8), lambda b: (b, 0, 0)),
        ],
        out_specs=pl.BlockSpec((1, PIX, PIX), lambda b: (b, 0, 0)),
        out_shape=jax.ShapeDtypeStruct((NIMG, PIX, PIX), jnp.float32),
        interpret=_INTERPRET,
    )(frow, fcol)

    s_row8 = jnp.broadcast_to(s.reshape(1, N), (8, N))

    t1t = pl.pallas_call(
        _t1_body,
        grid=(NBLK, NBLK),
        in_specs=[
            pl.BlockSpec((BLK, BLK), lambda j, ci: (j, ci)),
            pl.BlockSpec((BLK, BLK), lambda j, ci: (ci, j)),
            pl.BlockSpec((BLK, 1), lambda j, ci: (j, 0)),
            pl.BlockSpec((8, BLK), lambda j, ci: (0, ci)),
            pl.BlockSpec((1, PIX, PIX), lambda j, ci: (j, 0, 0)),
            pl.BlockSpec((BLK, KDIM), lambda j, ci: (ci, 0)),
        ],
        out_specs=pl.BlockSpec((BLK, KDIM), lambda j, ci: (j, 0)),
        out_shape=jax.ShapeDtypeStruct((N, KDIM), jnp.float32),
        interpret=_INTERPRET,
    )(res, res, s, s_row8, g2, psi_s)

    loss, reg = pl.pallas_call(
        _final_body,
        grid=(1,),
        in_specs=[
            pl.BlockSpec((N, KDIM), lambda i: (0, 0)),
            pl.BlockSpec((N, KDIM), lambda i: (0, 0)),
        ],
        out_specs=[
            pl.BlockSpec((1, 1), lambda i: (0, 0)),
            pl.BlockSpec((1, 1), lambda i: (0, 0)),
        ],
        out_shape=[
            jax.ShapeDtypeStruct((1, 1), jnp.float32),
            jax.ShapeDtypeStruct((1, 1), jnp.float32),
        ],
        interpret=_INTERPRET,
    )(t1t, psi_s)

    return (loss.reshape(()), reg.reshape(()))
